# baseline (speedup 1.0000x reference)
import functools

import jax
import jax.numpy as jnp
from jax import lax
from jax.experimental import pallas as pl
from jax.experimental.pallas import tpu as pltpu
from jax.experimental.pallas import tpu_sc as plsc

N = 100000
F = 9
V = 100
D = 128
L = 16
NC, NS = 2, 16
NW = NC * NS
ROWS_PER_TILE = 3200
NP = NW * ROWS_PER_TILE
CHUNK = 32
NCHUNK = ROWS_PER_TILE // CHUNK
IDX_PER_GATHER = 72
GPC = (CHUNK * F) // IDX_PER_GATHER
IDX_ROWS_PER_TILE = ROWS_PER_TILE * F // IDX_PER_GATHER


def _sc_encode(table_flat, idx2d):
    mesh = plsc.VectorSubcoreMesh(core_axis_name="c", subcore_axis_name="s")

    @functools.partial(
        pl.kernel,
        mesh=mesh,
        out_type=jax.ShapeDtypeStruct((NP, D), jnp.float32),
        scratch_types=[
            pltpu.VMEM((GPC, IDX_PER_GATHER), jnp.int32),
            pltpu.VMEM((CHUNK * F, D), jnp.float32),
            pltpu.VMEM((CHUNK, D), jnp.float32),
            pltpu.SemaphoreType.DMA,
        ],
    )
    def k(t_hbm, i_hbm, o_hbm, idx_v, rows_v, acc_v, sem):
        wid = lax.axis_index("s") * NC + lax.axis_index("c")

        @pl.loop(0, NCHUNK)
        def _chunk(c):
            row0 = wid * ROWS_PER_TILE + c * CHUNK
            ir0 = wid * IDX_ROWS_PER_TILE + c * GPC
            pltpu.sync_copy(i_hbm.at[pl.ds(ir0, GPC)], idx_v)
            copies = [
                pltpu.async_copy(
                    t_hbm.at[idx_v.at[j]],
                    rows_v.at[pl.ds(j * IDX_PER_GATHER, IDX_PER_GATHER)],
                    sem,
                )
                for j in range(GPC)
            ]
            for cp in copies:
                cp.wait()

            @pl.loop(0, CHUNK)
            def _row(r):
                base = r * F
                for d in range(D // L):
                    sl = pl.ds(d * L, L)
                    v = [rows_v[base + f, sl] for f in range(F)]
                    s01 = v[0] + v[1]
                    s23 = v[2] + v[3]
                    s45 = v[4] + v[5]
                    s67 = v[6] + v[7]
                    acc_v[r, sl] = ((s01 + s23) + (s45 + s67)) + v[8]

            pltpu.sync_copy(acc_v, o_hbm.at[pl.ds(row0, CHUNK)])

    return k(table_flat, idx2d)


def kernel(x, tables):
    if x.ndim == 1:
        x = x[:, None]
    n, f = x.shape
    v, d = tables.shape[1], tables.shape[2]
    table_flat = tables.reshape(f * v, d)
    flat_idx = (x + (jnp.arange(f, dtype=x.dtype) * v)[None, :]).reshape(-1)
    flat_idx = jnp.pad(flat_idx, (0, NP * F - n * f))
    idx2d = flat_idx.reshape(-1, IDX_PER_GATHER)
    out = _sc_encode(table_flat, idx2d)
    return out[:n]


# speedup vs baseline: 1.1543x; 1.1543x over previous
import functools

import jax
import jax.numpy as jnp
from jax import lax
from jax.experimental import pallas as pl
from jax.experimental.pallas import tpu as pltpu
from jax.experimental.pallas import tpu_sc as plsc

N = 100000
F = 9
V = 100
D = 128
L = 16
NC, NS = 2, 16
NW = NC * NS
ROWS_PER_TILE = 3200
NP = NW * ROWS_PER_TILE
CHUNK = 32
NCHUNK = ROWS_PER_TILE // CHUNK
IDX_PER_GATHER = 72
GPC = (CHUNK * F) // IDX_PER_GATHER
IDX_ROWS_PER_TILE = ROWS_PER_TILE * F // IDX_PER_GATHER


def _sc_encode(table_flat, idx2d):
    mesh = plsc.VectorSubcoreMesh(core_axis_name="c", subcore_axis_name="s")

    @functools.partial(
        pl.kernel,
        mesh=mesh,
        out_type=jax.ShapeDtypeStruct((NP, D), jnp.float32),
        scratch_types=[
            pltpu.VMEM((ROWS_PER_TILE * F,), jnp.int32),
            pltpu.VMEM((2, CHUNK * F, D), jnp.float32),
            pltpu.VMEM((2, CHUNK, D), jnp.float32),
            pltpu.SemaphoreType.DMA,
            pltpu.SemaphoreType.DMA,
            pltpu.SemaphoreType.DMA,
            pltpu.SemaphoreType.DMA,
            pltpu.SemaphoreType.DMA,
        ],
    )
    def k(t_hbm, i_hbm, o_hbm, idx_v, rows_v, acc_v,
          isem, gsem0, gsem1, osem0, osem1):
        gsems = (gsem0, gsem1)
        osems = (osem0, osem1)
        wid = lax.axis_index("s") * NC + lax.axis_index("c")
        row_base = wid * ROWS_PER_TILE

        pltpu.async_copy(
            i_hbm.at[pl.ds(wid * (ROWS_PER_TILE * F), ROWS_PER_TILE * F)],
            idx_v, isem,
        ).wait()

        def fire_gathers(cc, b):
            for j in range(GPC):
                off = pl.multiple_of((cc * GPC + j) * IDX_PER_GATHER, 8)
                pltpu.make_async_copy(
                    t_hbm.at[idx_v.at[pl.ds(off, IDX_PER_GATHER)]],
                    rows_v.at[b].at[pl.ds(j * IDX_PER_GATHER, IDX_PER_GATHER)],
                    gsems[b],
                ).start()

        def drain_gathers(b):
            for j in range(GPC):
                pltpu.make_async_copy(
                    t_hbm.at[idx_v.at[pl.ds(j * IDX_PER_GATHER, IDX_PER_GATHER)]],
                    rows_v.at[b].at[pl.ds(j * IDX_PER_GATHER, IDX_PER_GATHER)],
                    gsems[b],
                ).wait()

        def out_copy(cc, b):
            return pltpu.make_async_copy(
                acc_v.at[b],
                o_hbm.at[pl.ds(row_base + cc * CHUNK, CHUNK)],
                osems[b],
            )

        fire_gathers(0, 0)

        @pl.loop(0, NCHUNK, step=2)
        def _pair(c):
            for b in range(2):
                cc = c + b

                @pl.when(cc + 1 < NCHUNK)
                def _prefetch():
                    fire_gathers(cc + 1, 1 - b)

                drain_gathers(b)

                @pl.when(cc >= 2)
                def _reclaim_acc():
                    out_copy(cc - 2, b).wait()

                rows_b = rows_v.at[b]
                acc_b = acc_v.at[b]

                @pl.loop(0, CHUNK)
                def _row(r):
                    base = r * F
                    for d in range(D // L):
                        sl = pl.ds(d * L, L)
                        vv = [rows_b[base + f, sl] for f in range(F)]
                        s01 = vv[0] + vv[1]
                        s23 = vv[2] + vv[3]
                        s45 = vv[4] + vv[5]
                        s67 = vv[6] + vv[7]
                        acc_b[r, sl] = ((s01 + s23) + (s45 + s67)) + vv[8]

                out_copy(cc, b).start()

        out_copy(NCHUNK - 2, 0).wait()
        out_copy(NCHUNK - 1, 1).wait()

    return k(table_flat, idx2d)


def kernel(x, tables):
    if x.ndim == 1:
        x = x[:, None]
    n, f = x.shape
    v, d = tables.shape[1], tables.shape[2]
    table_flat = tables.reshape(f * v, d)
    flat_idx = (x + (jnp.arange(f, dtype=x.dtype) * v)[None, :]).reshape(-1)
    flat_idx = jnp.pad(flat_idx, (0, NP * F - n * f))
    out = _sc_encode(table_flat, flat_idx)
    return out[:n]


# speedup vs baseline: 4.0452x; 3.5046x over previous
import dataclasses
import functools

import jax
import jax.numpy as jnp
from jax import lax
from jax.experimental import pallas as pl
from jax.experimental.pallas import tpu as pltpu
from jax.experimental.pallas import tpu_sc as plsc

N = 100000
F = 9
V = 100
D = 128
L = 16
NC, NS = 2, 16
NW = NC * NS
ROWS_PER_TILE = 3200
NP = NW * ROWS_PER_TILE
CHUNK = 40
NCHUNK = ROWS_PER_TILE // CHUNK
IDX_W = 16
TROWS = F * V

_TAKE_DN = lax.GatherDimensionNumbers(
    offset_dims=(), collapsed_slice_dims=(0,), start_index_map=(0,))


def _vec_take(vec, idx):
    return lax.gather(vec, idx[:, None], _TAKE_DN, slice_sizes=(1,),
                      mode=lax.GatherScatterMode.PROMISE_IN_BOUNDS)


def _sc_encode(table_flat, idx_pad):
    mesh = plsc.VectorSubcoreMesh(core_axis_name="c", subcore_axis_name="s")
    cp = pltpu.CompilerParams()
    if "needs_layout_passes" in pltpu.CompilerParams.__dataclass_fields__:
        cp = dataclasses.replace(cp, needs_layout_passes=False)

    @functools.partial(
        pl.kernel,
        mesh=mesh,
        compiler_params=cp,
        out_type=jax.ShapeDtypeStruct((NP, D), jnp.float32),
        scratch_types=[
            pltpu.VMEM((TROWS, D), jnp.float32),
            pltpu.VMEM((2 * CHUNK * IDX_W,), jnp.int32),
            pltpu.VMEM((2, CHUNK, D), jnp.float32),
            pltpu.SemaphoreType.DMA,
            pltpu.SemaphoreType.DMA,
            pltpu.SemaphoreType.DMA,
            pltpu.SemaphoreType.DMA,
            pltpu.SemaphoreType.DMA,
        ],
    )
    def k(t_hbm, i_hbm, o_hbm, table_v, idx_v, acc_v,
          tsem, isem0, isem1, osem0, osem1):
        isems = (isem0, isem1)
        osems = (osem0, osem1)
        wid = lax.axis_index("s") * NC + lax.axis_index("c")
        row_base = wid * ROWS_PER_TILE
        idx_base = wid * (ROWS_PER_TILE * IDX_W)

        pltpu.async_copy(t_hbm, table_v, tsem).wait()

        def idx_copy(cc, b):
            off = pl.multiple_of(idx_base + cc * (CHUNK * IDX_W), 8)
            return pltpu.make_async_copy(
                i_hbm.at[pl.ds(off, CHUNK * IDX_W)],
                idx_v.at[pl.ds(b * (CHUNK * IDX_W), CHUNK * IDX_W)],
                isems[b],
            )

        def out_copy(cc, b):
            return pltpu.make_async_copy(
                acc_v.at[b],
                o_hbm.at[pl.ds(row_base + cc * CHUNK, CHUNK)],
                osems[b],
            )

        idx_copy(0, 0).start()
        idx_copy(1, 1).start()

        cols = [d * L + lax.iota(jnp.int32, L) for d in range(D // L)]

        @pl.loop(0, NCHUNK, step=2)
        def _pair(c):
            for b in range(2):
                cc = c + b
                idx_copy(cc, b).wait()

                @pl.when(cc >= 2)
                def _reclaim_acc():
                    out_copy(cc - 2, b).wait()

                acc_b = acc_v.at[b]
                ibase = b * (CHUNK * IDX_W)

                @pl.loop(0, CHUNK)
                def _row(r):
                    idx_vec = idx_v[pl.ds(ibase + r * IDX_W, IDX_W)]
                    sp = [
                        _vec_take(idx_vec, jnp.full((L,), f, dtype=jnp.int32))
                        for f in range(F)
                    ]
                    for d in range(D // L):
                        vv = [plsc.load_gather(table_v, [sp[f], cols[d]])
                              for f in range(F)]
                        s01 = vv[0] + vv[1]
                        s23 = vv[2] + vv[3]
                        s45 = vv[4] + vv[5]
                        s67 = vv[6] + vv[7]
                        acc_b[r, pl.ds(d * L, L)] = \
                            ((s01 + s23) + (s45 + s67)) + vv[8]

                @pl.when(cc + 2 < NCHUNK)
                def _prefetch_idx():
                    idx_copy(cc + 2, b).start()

                out_copy(cc, b).start()

        out_copy(NCHUNK - 2, 0).wait()
        out_copy(NCHUNK - 1, 1).wait()

    return k(table_flat, idx_pad)


def kernel(x, tables):
    if x.ndim == 1:
        x = x[:, None]
    n, f = x.shape
    v, d = tables.shape[1], tables.shape[2]
    table_flat = tables.reshape(f * v, d)
    flat_idx = x + (jnp.arange(f, dtype=x.dtype) * v)[None, :]
    idx_pad = jnp.zeros((NP, IDX_W), dtype=jnp.int32)
    idx_pad = idx_pad.at[:n, :f].set(flat_idx).reshape(-1)
    out = _sc_encode(table_flat, idx_pad)
    return out[:n]
